# baseline (speedup 1.0000x reference)
import functools

import jax
import jax.numpy as jnp
from jax import lax
from jax.experimental import pallas as pl
from jax.experimental.pallas import tpu as pltpu
from jax.experimental.pallas import tpu_sc as plsc

_N = 10000
_E = 320000
_D = 128
_G = 64
_NC = 2
_NS = 16
_K = 128
_T = 80
_EP = _NC * _NS * _T * _K
_PH = 40
_NPAD = _N + 16
_RPT = 624


def _sc_agg_body(z_hbm, srcb_hbm, dstb_hbm, zeros_hbm, out_hbm,
                 src_v, dst_v, rows0_v, rows1_v, agg_sh,
                 gsem0, gsem1, ssem0, ssem1):
    c = lax.axis_index("c")
    s = lax.axis_index("s")
    blk = c * _NS + s
    row0 = s * _RPT

    def _rowwise(fn):
        @pl.when(s < _NS - 1)
        def _():
            fn(row0, _RPT)

        @pl.when(s == _NS - 1)
        def _():
            fn((_NS - 1) * _RPT, _N - (_NS - 1) * _RPT)

    init_src = lambda r0, n: pltpu.sync_copy(
        z_hbm.at[pl.ds(r0, n)], agg_sh.at[pl.ds(r0, n)])
    init_zero = lambda r0, n: pltpu.sync_copy(
        zeros_hbm.at[pl.ds(r0, n)], agg_sh.at[pl.ds(r0, n)])

    @pl.when(c == 0)
    def _():
        _rowwise(init_src)

    @pl.when(c != 0)
    def _():
        _rowwise(init_zero)

    plsc.subcore_barrier()

    rows = (rows0_v, rows1_v)
    gsem = (gsem0, gsem1)
    ssem = (ssem0, ssem1)

    def _gather(t, q):
        pltpu.async_copy(z_hbm.at[src_v.at[t]], rows[q], gsem[q])

    def _gwait(q):
        pltpu.make_async_copy(z_hbm.at[src_v.at[0]], rows[q], gsem[q]).wait()

    def _scatter(t, q):
        pltpu.async_copy(rows[q], agg_sh.at[dst_v.at[t]], ssem[q], add=True)

    def _swait(q):
        pltpu.make_async_copy(rows[q], agg_sh.at[dst_v.at[0]], ssem[q]).wait()

    for ph in range(_T // _PH):
        c0 = ph * _PH
        pltpu.sync_copy(srcb_hbm.at[blk, pl.ds(c0, _PH)], src_v)
        pltpu.sync_copy(dstb_hbm.at[blk, pl.ds(c0, _PH)], dst_v)
        _gather(0, 0)
        _gwait(0)
        _scatter(0, 0)
        _gather(1, 1)

        def chunk_pair(i, carry):
            a = 2 * i + 1
            _gwait(1)
            _scatter(a, 1)
            _swait(0)
            _gather(a + 1, 0)
            _gwait(0)
            _scatter(a + 1, 0)
            _swait(1)

            @pl.when(a + 2 < _PH)
            def _():
                _gather(a + 2, 1)

            return carry

        lax.fori_loop(0, (_PH - 1) // 2, chunk_pair, 0)
        _gwait(1)
        _scatter(_PH - 1, 1)
        _swait(0)
        _swait(1)

    plsc.subcore_barrier()
    _rowwise(lambda r0, n: pltpu.sync_copy(
        agg_sh.at[pl.ds(r0, n)], out_hbm.at[c, pl.ds(r0, n)]))


@functools.lru_cache(maxsize=None)
def _make_sc_agg():
    return pl.kernel(
        _sc_agg_body,
        out_type=jax.ShapeDtypeStruct((_NC, _N, _D), jnp.float32),
        mesh=plsc.VectorSubcoreMesh(core_axis_name="c", subcore_axis_name="s"),
        scratch_types=[
            pltpu.VMEM((_PH, _K), jnp.int32),
            pltpu.VMEM((_PH, _K), jnp.int32),
            pltpu.VMEM((_K, _D), jnp.float32),
            pltpu.VMEM((_K, _D), jnp.float32),
            pltpu.VMEM_SHARED((_NPAD, _D), jnp.float32),
            pltpu.SemaphoreType.DMA,
            pltpu.SemaphoreType.DMA,
            pltpu.SemaphoreType.DMA,
            pltpu.SemaphoreType.DMA,
        ],
    )


def _tc_layer_body(agg_ref, w1_ref, b1_ref, w2_ref, b2_ref, gam_ref, bet_ref,
                   batch_ref, z_ref, g_ref):
    h = agg_ref[0] + agg_ref[1]
    h = jnp.maximum(
        jnp.dot(h, w1_ref[...], preferred_element_type=jnp.float32, precision=lax.Precision.HIGHEST) + b1_ref[...],
        0.0)
    h = jnp.dot(h, w2_ref[...], preferred_element_type=jnp.float32, precision=lax.Precision.HIGHEST) + b2_ref[...]
    h = jnp.maximum(h, 0.0)
    mean = jnp.mean(h, axis=0, keepdims=True)
    cen = h - mean
    var = jnp.mean(cen * cen, axis=0, keepdims=True)
    z = cen * (gam_ref[...] * lax.rsqrt(var + 1e-5)) + bet_ref[...]
    z_ref[...] = z
    onehot = (batch_ref[...] ==
              lax.broadcasted_iota(jnp.int32, (_G, _N), 0)).astype(jnp.float32)
    g_ref[...] = jnp.dot(onehot, z, preferred_element_type=jnp.float32, precision=lax.Precision.HIGHEST)


def _tc_layer(agg, w1, b1, w2, b2, gamma, beta, batch_row):
    return pl.pallas_call(
        _tc_layer_body,
        out_shape=(
            jax.ShapeDtypeStruct((_N, _D), jnp.float32),
            jax.ShapeDtypeStruct((_G, _D), jnp.float32),
        ),
    )(agg, w1, b1, w2, b2, gamma, beta, batch_row)


def kernel(x, edge_index, batch, W1_0, b1_0, W2_0, b2_0, gamma_0, beta_0,
           W1_1, b1_1, W2_1, b2_1, gamma_1, beta_1):
    pad = _EP - _E
    srcb = jnp.concatenate(
        [edge_index[0], jnp.zeros((pad,), jnp.int32)]).reshape(_NC * _NS, _T, _K)
    dstb = jnp.concatenate(
        [edge_index[1], jnp.full((pad,), _N, jnp.int32)]).reshape(_NC * _NS, _T, _K)
    zeros = jnp.zeros((_N, _D), jnp.float32)
    batch_row = batch.reshape(1, _N)

    z = x
    zs, gs = [], []
    for (w1, b1, w2, b2, gam, bet) in (
            (W1_0, b1_0, W2_0, b2_0, gamma_0, beta_0),
            (W1_1, b1_1, W2_1, b2_1, gamma_1, beta_1)):
        agg = _make_sc_agg()(z, srcb, dstb, zeros)
        z, g = _tc_layer(agg, w1, b1.reshape(1, _D), w2, b2.reshape(1, _D),
                         gam.reshape(1, _D), bet.reshape(1, _D), batch_row)
        zs.append(z)
        gs.append(g)
    return jnp.concatenate(zs, axis=1), jnp.concatenate(gs, axis=1)


# speedup vs baseline: 3.0713x; 3.0713x over previous
import functools

import jax
import jax.numpy as jnp
from jax import lax
from jax.experimental import pallas as pl
from jax.experimental.pallas import tpu as pltpu
from jax.experimental.pallas import tpu_sc as plsc

_N = 10000
_E = 320000
_D = 128
_G = 64
_NC = 2
_NS = 16
_K = 128
_T = 80
_EP = _NC * _NS * _T * _K
_PH = 40
_NZ = _N + 16
_RPT = 624


def _sc_agg_body(z_hbm, srcb_hbm, dstb_hbm, zeros_hbm, out_hbm,
                 src_v, dst_v, rows0_v, rows1_v, agg_sh,
                 gsem0, gsem1, ssem0, ssem1):
    c = lax.axis_index("c")
    s = lax.axis_index("s")
    blk = c * _NS + s
    row0 = s * _RPT

    def _rowwise(fn):
        @pl.when(s < _NS - 1)
        def _():
            fn(row0, _RPT)

        @pl.when(s == _NS - 1)
        def _():
            fn((_NS - 1) * _RPT, _N - (_NS - 1) * _RPT)

    init_src = lambda r0, n: pltpu.sync_copy(
        z_hbm.at[pl.ds(r0, n)], agg_sh.at[pl.ds(r0, n)])
    init_zero = lambda r0, n: pltpu.sync_copy(
        zeros_hbm.at[pl.ds(r0, n)], agg_sh.at[pl.ds(r0, n)])

    @pl.when(c == 0)
    def _():
        _rowwise(init_src)

    @pl.when(c != 0)
    def _():
        _rowwise(init_zero)

    plsc.subcore_barrier()

    rows = (rows0_v, rows1_v)
    gsem = (gsem0, gsem1)
    ssem = (ssem0, ssem1)

    def _gather(t, q):
        pltpu.async_copy(z_hbm.at[src_v.at[t]], rows[q], gsem[q])

    def _gwait(q):
        pltpu.make_async_copy(z_hbm.at[src_v.at[0]], rows[q], gsem[q]).wait()

    def _scatter(t, q):
        pltpu.async_copy(rows[q], agg_sh.at[dst_v.at[t]], ssem[q], add=True)

    def _swait(q):
        pltpu.make_async_copy(rows[q], agg_sh.at[dst_v.at[0]], ssem[q]).wait()

    for ph in range(_T // _PH):
        c0 = ph * _PH
        pltpu.sync_copy(srcb_hbm.at[blk, pl.ds(c0, _PH)], src_v)
        pltpu.sync_copy(dstb_hbm.at[blk, pl.ds(c0, _PH)], dst_v)
        _gather(0, 0)
        _gwait(0)
        _scatter(0, 0)
        _gather(1, 1)

        def chunk_pair(i, carry):
            a = 2 * i + 1
            _gwait(1)
            _scatter(a, 1)
            _swait(0)
            _gather(a + 1, 0)
            _gwait(0)
            _scatter(a + 1, 0)
            _swait(1)

            @pl.when(a + 2 < _PH)
            def _():
                _gather(a + 2, 1)

            return carry

        lax.fori_loop(0, (_PH - 1) // 2, chunk_pair, 0)
        _gwait(1)
        _scatter(_PH - 1, 1)
        _swait(0)
        _swait(1)

    plsc.subcore_barrier()
    _rowwise(lambda r0, n: pltpu.sync_copy(
        agg_sh.at[pl.ds(r0, n)], out_hbm.at[c, pl.ds(r0, n)]))


@functools.lru_cache(maxsize=None)
def _make_sc_agg():
    return pl.kernel(
        _sc_agg_body,
        out_type=jax.ShapeDtypeStruct((_NC, _N, _D), jnp.float32),
        mesh=plsc.VectorSubcoreMesh(core_axis_name="c", subcore_axis_name="s"),
        scratch_types=[
            pltpu.VMEM((_PH, _K), jnp.int32),
            pltpu.VMEM((_PH, _K), jnp.int32),
            pltpu.VMEM((_K, _D), jnp.float32),
            pltpu.VMEM((_K, _D), jnp.float32),
            pltpu.VMEM_SHARED((_N, _D), jnp.float32),
            pltpu.SemaphoreType.DMA,
            pltpu.SemaphoreType.DMA,
            pltpu.SemaphoreType.DMA,
            pltpu.SemaphoreType.DMA,
        ],
    )


def _tc_layer_body(agg_ref, w1_ref, b1_ref, w2_ref, b2_ref, gam_ref, bet_ref,
                   batch_ref, z_ref, g_ref):
    h = agg_ref[0] + agg_ref[1]
    h = jnp.maximum(
        jnp.dot(h, w1_ref[...], preferred_element_type=jnp.float32, precision=lax.Precision.HIGHEST) + b1_ref[...],
        0.0)
    h = jnp.dot(h, w2_ref[...], preferred_element_type=jnp.float32, precision=lax.Precision.HIGHEST) + b2_ref[...]
    h = jnp.maximum(h, 0.0)
    mean = jnp.mean(h, axis=0, keepdims=True)
    cen = h - mean
    var = jnp.mean(cen * cen, axis=0, keepdims=True)
    z = cen * (gam_ref[...] * lax.rsqrt(var + 1e-5)) + bet_ref[...]
    z_ref[...] = z
    onehot = (batch_ref[...] ==
              lax.broadcasted_iota(jnp.int32, (_G, _N), 0)).astype(jnp.float32)
    g_ref[...] = jnp.dot(onehot, z, preferred_element_type=jnp.float32, precision=lax.Precision.HIGHEST)


def _tc_layer(agg, w1, b1, w2, b2, gamma, beta, batch_row):
    return pl.pallas_call(
        _tc_layer_body,
        out_shape=(
            jax.ShapeDtypeStruct((_N, _D), jnp.float32),
            jax.ShapeDtypeStruct((_G, _D), jnp.float32),
        ),
    )(agg, w1, b1, w2, b2, gamma, beta, batch_row)


def kernel(x, edge_index, batch, W1_0, b1_0, W2_0, b2_0, gamma_0, beta_0,
           W1_1, b1_1, W2_1, b2_1, gamma_1, beta_1):
    pad = _EP - _E
    pr = jnp.arange(pad, dtype=jnp.int32)
    srcb = jnp.concatenate(
        [edge_index[0], _N + pr % 16]).reshape(_NC * _NS, _T, _K)
    dstb = jnp.concatenate(
        [edge_index[1], pr % _N]).reshape(_NC * _NS, _T, _K)
    zrows = jnp.zeros((16, _D), jnp.float32)
    zeros = jnp.zeros((_N, _D), jnp.float32)
    batch_row = batch.reshape(1, _N)

    z = x
    zs, gs = [], []
    for (w1, b1, w2, b2, gam, bet) in (
            (W1_0, b1_0, W2_0, b2_0, gamma_0, beta_0),
            (W1_1, b1_1, W2_1, b2_1, gamma_1, beta_1)):
        agg = _make_sc_agg()(jnp.concatenate([z, zrows]), srcb, dstb, zeros)
        z, g = _tc_layer(agg, w1, b1.reshape(1, _D), w2, b2.reshape(1, _D),
                         gam.reshape(1, _D), bet.reshape(1, _D), batch_row)
        zs.append(z)
        gs.append(g)
    return jnp.concatenate(zs, axis=1), jnp.concatenate(gs, axis=1)


# speedup vs baseline: 3.2738x; 1.0659x over previous
import functools

import jax
import jax.numpy as jnp
from jax import lax
from jax.experimental import pallas as pl
from jax.experimental.pallas import tpu as pltpu
from jax.experimental.pallas import tpu_sc as plsc

_N = 10000
_E = 320000
_D = 128
_G = 64
_NC = 2
_NS = 16
_K = 128
_T = 80
_EP = _NC * _NS * _T * _K
_PH = 40
_NZ = _N + 16
_RPT = 624


def _sc_agg_body(z_hbm, srcb_hbm, dstb_hbm, zeros_hbm, out_hbm,
                 src_v, dst_v, rows0_v, rows1_v, agg_sh,
                 gsem0, gsem1, ssem0, ssem1):
    c = lax.axis_index("c")
    s = lax.axis_index("s")
    blk = c * _NS + s
    row0 = s * _RPT

    def _rowwise(fn):
        @pl.when(s < _NS - 1)
        def _():
            fn(row0, _RPT)

        @pl.when(s == _NS - 1)
        def _():
            fn((_NS - 1) * _RPT, _N - (_NS - 1) * _RPT)

    init_src = lambda r0, n: pltpu.sync_copy(
        z_hbm.at[pl.ds(r0, n)], agg_sh.at[pl.ds(r0, n)])
    init_zero = lambda r0, n: pltpu.sync_copy(
        zeros_hbm.at[pl.ds(r0, n)], agg_sh.at[pl.ds(r0, n)])

    @pl.when(c == 0)
    def _():
        _rowwise(init_src)

    @pl.when(c != 0)
    def _():
        _rowwise(init_zero)

    plsc.subcore_barrier()

    rows = (rows0_v, rows1_v)
    gsem = (gsem0, gsem1)
    ssem = (ssem0, ssem1)

    def _gather(t, q):
        pltpu.async_copy(z_hbm.at[src_v.at[t]], rows[q], gsem[q])

    def _gwait(q):
        pltpu.make_async_copy(z_hbm.at[src_v.at[0]], rows[q], gsem[q]).wait()

    def _scatter(t, q):
        pltpu.async_copy(rows[q], agg_sh.at[dst_v.at[t]], ssem[q], add=True)

    def _swait(q):
        pltpu.make_async_copy(rows[q], agg_sh.at[dst_v.at[0]], ssem[q]).wait()

    for ph in range(_T // _PH):
        c0 = ph * _PH
        pltpu.sync_copy(srcb_hbm.at[blk, pl.ds(c0, _PH)], src_v)
        pltpu.sync_copy(dstb_hbm.at[blk, pl.ds(c0, _PH)], dst_v)
        _gather(0, 0)
        _gwait(0)
        _scatter(0, 0)
        _gather(1, 1)

        def chunk_pair(i, carry):
            a = 2 * i + 1
            _gwait(1)
            _scatter(a, 1)
            _swait(0)
            _gather(a + 1, 0)
            _gwait(0)
            _scatter(a + 1, 0)
            _swait(1)

            @pl.when(a + 2 < _PH)
            def _():
                _gather(a + 2, 1)

            return carry

        lax.fori_loop(0, (_PH - 1) // 2, chunk_pair, 0)
        _gwait(1)
        _scatter(_PH - 1, 1)
        _swait(0)
        _swait(1)

    plsc.subcore_barrier()
    _rowwise(lambda r0, n: pltpu.sync_copy(
        agg_sh.at[pl.ds(r0, n)], out_hbm.at[c, pl.ds(r0, n)]))


@functools.lru_cache(maxsize=None)
def _make_sc_agg():
    return pl.kernel(
        _sc_agg_body,
        out_type=jax.ShapeDtypeStruct((_NC, _N, _D), jnp.float32),
        mesh=plsc.VectorSubcoreMesh(core_axis_name="c", subcore_axis_name="s"),
        scratch_types=[
            pltpu.VMEM((_PH, _K), jnp.int32),
            pltpu.VMEM((_PH, _K), jnp.int32),
            pltpu.VMEM((_K, _D), jnp.float32),
            pltpu.VMEM((_K, _D), jnp.float32),
            pltpu.VMEM_SHARED((_N, _D), jnp.float32),
            pltpu.SemaphoreType.DMA,
            pltpu.SemaphoreType.DMA,
            pltpu.SemaphoreType.DMA,
            pltpu.SemaphoreType.DMA,
        ],
    )


def _tc_compute(agg_ref, w1_ref, b1_ref, w2_ref, b2_ref, gam_ref, bet_ref,
                batch_ref):
    h = agg_ref[0] + agg_ref[1]
    h = jnp.maximum(
        jnp.dot(h, w1_ref[...], preferred_element_type=jnp.float32,
                precision=lax.Precision.HIGHEST) + b1_ref[...], 0.0)
    h = jnp.dot(h, w2_ref[...], preferred_element_type=jnp.float32,
                precision=lax.Precision.HIGHEST) + b2_ref[...]
    h = jnp.maximum(h, 0.0)
    mean = jnp.mean(h, axis=0, keepdims=True)
    cen = h - mean
    var = jnp.mean(cen * cen, axis=0, keepdims=True)
    z = cen * (gam_ref[...] * lax.rsqrt(var + 1e-5)) + bet_ref[...]
    onehot = (batch_ref[...] ==
              lax.broadcasted_iota(jnp.int32, (_G, _N), 0)).astype(jnp.float32)
    g = jnp.dot(onehot, z, preferred_element_type=jnp.float32,
                precision=lax.Precision.HIGHEST)
    return z, g


def _tc_layer1_body(agg_ref, w1_ref, b1_ref, w2_ref, b2_ref, gam_ref, bet_ref,
                    batch_ref, zp_ref, g_ref):
    z, g = _tc_compute(agg_ref, w1_ref, b1_ref, w2_ref, b2_ref, gam_ref,
                       bet_ref, batch_ref)
    zp_ref[pl.ds(0, _N), :] = z
    zp_ref[pl.ds(_N, _NZ - _N), :] = jnp.zeros((_NZ - _N, _D), jnp.float32)
    g_ref[...] = g


def _tc_layer2_body(agg_ref, w1_ref, b1_ref, w2_ref, b2_ref, gam_ref, bet_ref,
                    batch_ref, zp1_ref, g1_ref, zcat_ref, gcat_ref):
    z2, g2 = _tc_compute(agg_ref, w1_ref, b1_ref, w2_ref, b2_ref, gam_ref,
                         bet_ref, batch_ref)
    zcat_ref[:, pl.ds(0, _D)] = zp1_ref[pl.ds(0, _N), :]
    zcat_ref[:, pl.ds(_D, _D)] = z2
    gcat_ref[:, pl.ds(0, _D)] = g1_ref[...]
    gcat_ref[:, pl.ds(_D, _D)] = g2


def _tc_layer1(agg, w1, b1, w2, b2, gamma, beta, batch_row):
    return pl.pallas_call(
        _tc_layer1_body,
        out_shape=(
            jax.ShapeDtypeStruct((_NZ, _D), jnp.float32),
            jax.ShapeDtypeStruct((_G, _D), jnp.float32),
        ),
    )(agg, w1, b1, w2, b2, gamma, beta, batch_row)


def _tc_layer2(agg, w1, b1, w2, b2, gamma, beta, batch_row, zp1, g1):
    return pl.pallas_call(
        _tc_layer2_body,
        out_shape=(
            jax.ShapeDtypeStruct((_N, 2 * _D), jnp.float32),
            jax.ShapeDtypeStruct((_G, 2 * _D), jnp.float32),
        ),
    )(agg, w1, b1, w2, b2, gamma, beta, batch_row, zp1, g1)


def kernel(x, edge_index, batch, W1_0, b1_0, W2_0, b2_0, gamma_0, beta_0,
           W1_1, b1_1, W2_1, b2_1, gamma_1, beta_1):
    pad = _EP - _E
    pr = jnp.arange(pad, dtype=jnp.int32)
    srcb = jnp.concatenate(
        [edge_index[0], _N + pr % 16]).reshape(_NC * _NS, _T, _K)
    dstb = jnp.concatenate(
        [edge_index[1], pr % _N]).reshape(_NC * _NS, _T, _K)
    zeros = jnp.zeros((_N, _D), jnp.float32)
    batch_row = batch.reshape(1, _N)
    xp = jnp.concatenate([x, jnp.zeros((_NZ - _N, _D), jnp.float32)])

    agg0 = _make_sc_agg()(xp, srcb, dstb, zeros)
    zp1, g1 = _tc_layer1(agg0, W1_0, b1_0.reshape(1, _D), W2_0,
                         b2_0.reshape(1, _D), gamma_0.reshape(1, _D),
                         beta_0.reshape(1, _D), batch_row)
    agg1 = _make_sc_agg()(zp1, srcb, dstb, zeros)
    return _tc_layer2(agg1, W1_1, b1_1.reshape(1, _D), W2_1,
                      b2_1.reshape(1, _D), gamma_1.reshape(1, _D),
                      beta_1.reshape(1, _D), batch_row, zp1, g1)
